# Initial kernel scaffold; baseline (speedup 1.0000x reference)
import jax
import jax.numpy as jnp
from jax.experimental import pallas as pl


def kernel(means3D, means2D, opacities, colors_precomp, scales, rotations, theta, rho, viewmatrix, bg):
    raise NotImplementedError("write your pallas kernel here")


# speedup vs baseline: 2.3810x; 2.3810x over previous
import functools

import jax
import jax.numpy as jnp
from jax.experimental import pallas as pl
from jax.experimental.pallas import tpu as pltpu

H = 64
W = 64
N = 2048
TANX = 0.5
TANY = 0.5
SCALE_MOD = 1.0

CHUNK = 256
PIX_TILE = 512
NCHUNK = N // CHUNK
NPIX = H * W

_HI = jax.lax.Precision.HIGHEST


def _rp(x):
    i = jax.lax.bitcast_convert_type(x, jnp.int32)
    i = i + jnp.int32(0x7FFF) + ((i >> 16) & jnp.int32(1))
    i = i & jnp.int32(-65536)
    return jax.lax.bitcast_convert_type(i, jnp.float32)


def _geom_kernel(pin_ref, vm_ref, vmr_ref, params_ref, radii_ref):
    def row(i):
        return pin_ref[i:i + 1, :]

    m0, m1, m2 = row(0), row(1), row(2)
    s = [row(3), row(4), row(5)]
    qr, qx, qy, qz = row(6), row(7), row(8), row(9)
    opac = row(10)
    col = [row(11), row(12), row(13)]

    qn = jnp.sqrt(qr * qr + qx * qx + qy * qy + qz * qz) + 1e-8
    r, x, y, z = qr / qn, qx / qn, qy / qn, qz / qn
    R = [
        [1 - 2 * (y * y + z * z), 2 * (x * y - r * z), 2 * (x * z + r * y)],
        [2 * (x * y + r * z), 1 - 2 * (x * x + z * z), 2 * (y * z - r * x)],
        [2 * (x * z - r * y), 2 * (y * z + r * x), 1 - 2 * (x * x + y * y)],
    ]
    L = [[_rp(R[a][c] * (s[c] * SCALE_MOD)) for c in range(3)] for a in range(3)]
    Sig = [[L[a][0] * L[b][0] + L[a][1] * L[b][1] + L[a][2] * L[b][2]
            for b in range(3)] for a in range(3)]

    V = [[vm_ref[a, b] for b in range(4)] for a in range(4)]
    Vr = [[vmr_ref[a, b] for b in range(4)] for a in range(4)]
    mr = [_rp(m0), _rp(m1), _rp(m2)]
    xv = mr[0] * Vr[0][0] + mr[1] * Vr[1][0] + mr[2] * Vr[2][0] + V[3][0]
    yv = mr[0] * Vr[0][1] + mr[1] * Vr[1][1] + mr[2] * Vr[2][1] + V[3][1]
    zv = mr[0] * Vr[0][2] + mr[1] * Vr[1][2] + mr[2] * Vr[2][2] + V[3][2]

    mask = zv > 0.2
    zc = jnp.maximum(zv, 0.2)
    fx = W / (2.0 * TANX)
    fy = H / (2.0 * TANY)
    limx = 1.3 * TANX
    limy = 1.3 * TANY
    tx = jnp.clip(xv / zc, -limx, limx) * zc
    ty = jnp.clip(yv / zc, -limy, limy) * zc
    j00 = fx / zc
    j02 = -fx * tx / (zc * zc)
    j11 = fy / zc
    j12 = -fy * ty / (zc * zc)

    Sr = [[_rp(Sig[a][b]) for b in range(3)] for a in range(3)]
    tmp = [[Vr[0][i] * Sr[0][k] + Vr[1][i] * Sr[1][k] + Vr[2][i] * Sr[2][k]
            for k in range(3)] for i in range(3)]
    tr = [[_rp(tmp[i][k]) for k in range(3)] for i in range(3)]
    M = [[tr[i][0] * Vr[0][l] + tr[i][1] * Vr[1][l] + tr[i][2] * Vr[2][l]
          for l in range(3)] for i in range(3)]

    Jr = [[_rp(j00), None, _rp(j02)], [None, _rp(j11), _rp(j12)]]
    Mr = [[_rp(M[a][b]) for b in range(3)] for a in range(3)]
    tmp2 = [[Jr[0][0] * Mr[0][k] + Jr[0][2] * Mr[2][k] for k in range(3)],
            [Jr[1][1] * Mr[1][k] + Jr[1][2] * Mr[2][k] for k in range(3)]]
    t2r = [[_rp(tmp2[i][k]) for k in range(3)] for i in range(2)]
    cov00 = t2r[0][0] * Jr[0][0] + t2r[0][2] * Jr[0][2]
    cov01 = t2r[0][1] * Jr[1][1] + t2r[0][2] * Jr[1][2]
    cov11 = t2r[1][1] * Jr[1][1] + t2r[1][2] * Jr[1][2]

    a_ = cov00 + 0.3
    b_ = cov01
    d_ = cov11 + 0.3
    det = jnp.maximum(a_ * d_ - b_ * b_, 1e-8)
    ca = d_ / det
    cb = -b_ / det
    cc = a_ / det
    mid = 0.5 * (a_ + d_)
    lam1 = mid + jnp.sqrt(jnp.maximum(mid * mid - det, 0.1))
    radii_ref[0:1, :] = jnp.where(
        mask, jnp.ceil(3.0 * jnp.sqrt(lam1)), 0.0).astype(jnp.int32)

    px = ((xv / zc) / TANX + 1.0) * W * 0.5 - 0.5
    py = ((yv / zc) / TANY + 1.0) * H * 0.5 - 0.5
    op_eff = jnp.where(mask, opac, 0.0)

    zero = jnp.zeros((1, N), jnp.float32)
    params_ref[...] = jnp.concatenate(
        [ca, cb, cc, px, py, op_eff, col[0], col[1], col[2], zc,
         zero, zero, zero, zero, zero, zero], axis=0)


def _sort_kernel(params_ref, zcol_ref, sorted_ref, rank_ref):
    zcol = zcol_ref[...]
    zrow = params_ref[9:10, :]

    rank = jnp.zeros((N, 1), jnp.float32)
    for c in range(NCHUNK):
        sl = slice(c * CHUNK, (c + 1) * CHUNK)
        zi = jnp.broadcast_to(zcol, (N, CHUNK))
        zj = jnp.broadcast_to(zrow[:, sl], (N, CHUNK))
        ii = jax.lax.broadcasted_iota(jnp.int32, (N, CHUNK), 0)
        jj = jax.lax.broadcasted_iota(jnp.int32, (N, CHUNK), 1) + c * CHUNK
        before = (zj < zi) | ((zj == zi) & (jj < ii))
        rank = rank + jnp.sum(before.astype(jnp.float32), axis=1, keepdims=True)
    rank_ref[...] = rank

    params = params_ref[...]
    for c in range(NCHUNK):
        kidx = (jax.lax.broadcasted_iota(jnp.int32, (N, CHUNK), 1)
                + c * CHUNK).astype(jnp.float32)
        onehot = (rank == kidx).astype(jnp.float32)
        sorted_ref[:, c * CHUNK:(c + 1) * CHUNK] = jnp.dot(
            params, onehot, preferred_element_type=jnp.float32, precision=_HI)


def _raster_kernel(sorted_ref, sortedT_ref, bg_ref, img_ref, nt_ref):
    t = pl.program_id(0)

    @pl.when(t == 0)
    def _init():
        nt_ref[...] = jnp.zeros_like(nt_ref)

    pix = jax.lax.broadcasted_iota(jnp.int32, (PIX_TILE, 1), 0) + t * PIX_TILE
    gx = (pix % W).astype(jnp.float32)
    gy = (pix // W).astype(jnp.float32)

    rj = jax.lax.broadcasted_iota(jnp.int32, (CHUNK, CHUNK), 0)
    rk = jax.lax.broadcasted_iota(jnp.int32, (CHUNK, CHUNK), 1)
    ut = (rj < rk).astype(jnp.float32)

    tcarry = jnp.ones((PIX_TILE, 1), jnp.float32)
    rgbd = jnp.zeros((PIX_TILE, 4), jnp.float32)
    for c in range(NCHUNK):
        sl = slice(c * CHUNK, (c + 1) * CHUNK)
        ca = sorted_ref[0:1, sl]
        cb = sorted_ref[1:2, sl]
        cc = sorted_ref[2:3, sl]
        px = sorted_ref[3:4, sl]
        py = sorted_ref[4:5, sl]
        op = sorted_ref[5:6, sl]
        c4 = sortedT_ref[sl, 6:10]

        dx = px - gx
        dy = py - gy
        power = -0.5 * (ca * dx * dx + cc * dy * dy) - cb * dx * dy
        g = jnp.exp(jnp.minimum(power, 0.0))
        alpha = jnp.minimum(0.99, op * g)
        valid = (power <= 0.0) & (alpha >= 1.0 / 255.0)
        alpha = jnp.where(valid, alpha, 0.0)

        logm = jnp.log1p(-alpha)
        sex = jnp.dot(logm, ut, preferred_element_type=jnp.float32,
                      precision=_HI)
        w = alpha * (tcarry * jnp.exp(sex))
        rgbd = rgbd + jnp.dot(w, c4, preferred_element_type=jnp.float32,
                              precision=_HI)
        tcarry = tcarry * jnp.exp(sex[:, CHUNK - 1:CHUNK] + logm[:, CHUNK - 1:CHUNK])

        nt_ref[0:1, sl] += jnp.sum(
            (alpha > 0.0).astype(jnp.float32), axis=0, keepdims=True)

    bg0 = bg_ref[0, 0]
    bg1 = bg_ref[0, 1]
    bg2 = bg_ref[0, 2]
    zeros3 = jnp.zeros((PIX_TILE, 3), jnp.float32)
    img_ref[...] = jnp.concatenate(
        [rgbd[:, 0:1] + tcarry * bg0,
         rgbd[:, 1:2] + tcarry * bg1,
         rgbd[:, 2:3] + tcarry * bg2,
         rgbd[:, 3:4],
         1.0 - tcarry,
         zeros3], axis=1)


def _unpermute_kernel(rank_ref, nt_ref, out_ref):
    rank = rank_ref[...]
    acc = jnp.zeros((N, 1), jnp.float32)
    for c in range(NCHUNK):
        kidx = (jax.lax.broadcasted_iota(jnp.int32, (N, CHUNK), 1)
                + c * CHUNK).astype(jnp.float32)
        eq = (rank == kidx).astype(jnp.float32)
        ntc = nt_ref[0:1, c * CHUNK:(c + 1) * CHUNK]
        acc = acc + jnp.sum(eq * ntc, axis=1, keepdims=True)
    out_ref[...] = acc.astype(jnp.int32)


@jax.jit
def kernel(means3D, means2D, opacities, colors_precomp, scales, rotations,
           theta, rho, viewmatrix, bg):
    del means2D, theta, rho
    pin = jnp.concatenate(
        [means3D.T.astype(jnp.float32),
         scales.T.astype(jnp.float32),
         rotations.T.astype(jnp.float32),
         opacities.T.astype(jnp.float32),
         colors_precomp.T.astype(jnp.float32),
         jnp.zeros((2, N), jnp.float32)], axis=0)

    vm32 = viewmatrix.astype(jnp.float32)
    params, radii2 = pl.pallas_call(
        _geom_kernel,
        out_shape=(
            jax.ShapeDtypeStruct((16, N), jnp.float32),
            jax.ShapeDtypeStruct((1, N), jnp.int32),
        ),
        in_specs=[
            pl.BlockSpec((16, N), lambda: (0, 0)),
            pl.BlockSpec(memory_space=pltpu.SMEM),
            pl.BlockSpec(memory_space=pltpu.SMEM),
        ],
    )(pin, vm32, _rp(vm32))

    zcol = params[9, :].reshape(N, 1)
    sorted_params, rank = pl.pallas_call(
        _sort_kernel,
        out_shape=(
            jax.ShapeDtypeStruct((16, N), jnp.float32),
            jax.ShapeDtypeStruct((N, 1), jnp.float32),
        ),
    )(params, zcol)

    img, nt_sorted = pl.pallas_call(
        _raster_kernel,
        grid=(NPIX // PIX_TILE,),
        out_shape=(
            jax.ShapeDtypeStruct((NPIX, 8), jnp.float32),
            jax.ShapeDtypeStruct((1, N), jnp.float32),
        ),
        in_specs=[
            pl.BlockSpec((16, N), lambda i: (0, 0)),
            pl.BlockSpec((N, 16), lambda i: (0, 0)),
            pl.BlockSpec(memory_space=pltpu.SMEM),
        ],
        out_specs=(
            pl.BlockSpec((PIX_TILE, 8), lambda i: (i, 0)),
            pl.BlockSpec((1, N), lambda i: (0, 0)),
        ),
    )(sorted_params, sorted_params.T, bg.reshape(1, 3).astype(jnp.float32))

    nt = pl.pallas_call(
        _unpermute_kernel,
        out_shape=jax.ShapeDtypeStruct((N, 1), jnp.int32),
    )(rank, nt_sorted)

    color = img[:, 0:3].T.reshape(3, H, W)
    depth = img[:, 3].reshape(1, H, W)
    opac = img[:, 4].reshape(1, H, W)
    return color, radii2.reshape(N), depth, opac, nt.reshape(N)


# speedup vs baseline: 2.9806x; 1.2518x over previous
import functools

import jax
import jax.numpy as jnp
from jax.experimental import pallas as pl
from jax.experimental.pallas import tpu as pltpu

H = 64
W = 64
N = 2048
TANX = 0.5
TANY = 0.5
SCALE_MOD = 1.0

CHUNK = 256
PIX_TILE = 512
NCHUNK = N // CHUNK
NPIX = H * W

_HI = jax.lax.Precision.HIGHEST


def _split2(x):
    hi = _rp(x)
    lo = _rp(x - hi)
    return hi, lo


def _rp(x):
    i = jax.lax.bitcast_convert_type(x, jnp.int32)
    i = i + jnp.int32(0x7FFF) + ((i >> 16) & jnp.int32(1))
    i = i & jnp.int32(-65536)
    return jax.lax.bitcast_convert_type(i, jnp.float32)


def _geom_kernel(pin_ref, vm_ref, vmr_ref, params_ref, radii_ref):
    def row(i):
        return pin_ref[i:i + 1, :]

    m0, m1, m2 = row(0), row(1), row(2)
    s = [row(3), row(4), row(5)]
    qr, qx, qy, qz = row(6), row(7), row(8), row(9)
    opac = row(10)
    col = [row(11), row(12), row(13)]

    qn = jnp.sqrt(qr * qr + qx * qx + qy * qy + qz * qz) + 1e-8
    r, x, y, z = qr / qn, qx / qn, qy / qn, qz / qn
    R = [
        [1 - 2 * (y * y + z * z), 2 * (x * y - r * z), 2 * (x * z + r * y)],
        [2 * (x * y + r * z), 1 - 2 * (x * x + z * z), 2 * (y * z - r * x)],
        [2 * (x * z - r * y), 2 * (y * z + r * x), 1 - 2 * (x * x + y * y)],
    ]
    L = [[_rp(R[a][c] * (s[c] * SCALE_MOD)) for c in range(3)] for a in range(3)]
    Sig = [[L[a][0] * L[b][0] + L[a][1] * L[b][1] + L[a][2] * L[b][2]
            for b in range(3)] for a in range(3)]

    V = [[vm_ref[a, b] for b in range(4)] for a in range(4)]
    Vr = [[vmr_ref[a, b] for b in range(4)] for a in range(4)]
    mr = [_rp(m0), _rp(m1), _rp(m2)]
    xv = mr[0] * Vr[0][0] + mr[1] * Vr[1][0] + mr[2] * Vr[2][0] + V[3][0]
    yv = mr[0] * Vr[0][1] + mr[1] * Vr[1][1] + mr[2] * Vr[2][1] + V[3][1]
    zv = mr[0] * Vr[0][2] + mr[1] * Vr[1][2] + mr[2] * Vr[2][2] + V[3][2]

    mask = zv > 0.2
    zc = jnp.maximum(zv, 0.2)
    fx = W / (2.0 * TANX)
    fy = H / (2.0 * TANY)
    limx = 1.3 * TANX
    limy = 1.3 * TANY
    tx = jnp.clip(xv / zc, -limx, limx) * zc
    ty = jnp.clip(yv / zc, -limy, limy) * zc
    j00 = fx / zc
    j02 = -fx * tx / (zc * zc)
    j11 = fy / zc
    j12 = -fy * ty / (zc * zc)

    Sr = [[_rp(Sig[a][b]) for b in range(3)] for a in range(3)]
    tmp = [[Vr[0][i] * Sr[0][k] + Vr[1][i] * Sr[1][k] + Vr[2][i] * Sr[2][k]
            for k in range(3)] for i in range(3)]
    tr = [[_rp(tmp[i][k]) for k in range(3)] for i in range(3)]
    M = [[tr[i][0] * Vr[0][l] + tr[i][1] * Vr[1][l] + tr[i][2] * Vr[2][l]
          for l in range(3)] for i in range(3)]

    Jr = [[_rp(j00), None, _rp(j02)], [None, _rp(j11), _rp(j12)]]
    Mr = [[_rp(M[a][b]) for b in range(3)] for a in range(3)]
    tmp2 = [[Jr[0][0] * Mr[0][k] + Jr[0][2] * Mr[2][k] for k in range(3)],
            [Jr[1][1] * Mr[1][k] + Jr[1][2] * Mr[2][k] for k in range(3)]]
    t2r = [[_rp(tmp2[i][k]) for k in range(3)] for i in range(2)]
    cov00 = t2r[0][0] * Jr[0][0] + t2r[0][2] * Jr[0][2]
    cov01 = t2r[0][1] * Jr[1][1] + t2r[0][2] * Jr[1][2]
    cov11 = t2r[1][1] * Jr[1][1] + t2r[1][2] * Jr[1][2]

    a_ = cov00 + 0.3
    b_ = cov01
    d_ = cov11 + 0.3
    det = jnp.maximum(a_ * d_ - b_ * b_, 1e-8)
    ca = d_ / det
    cb = -b_ / det
    cc = a_ / det
    mid = 0.5 * (a_ + d_)
    lam1 = mid + jnp.sqrt(jnp.maximum(mid * mid - det, 0.1))
    radii_ref[0:1, :] = jnp.where(
        mask, jnp.ceil(3.0 * jnp.sqrt(lam1)), 0.0).astype(jnp.int32)

    px = ((xv / zc) / TANX + 1.0) * W * 0.5 - 0.5
    py = ((yv / zc) / TANY + 1.0) * H * 0.5 - 0.5
    op_eff = jnp.where(mask, opac, 0.0)

    zero = jnp.zeros((1, N), jnp.float32)
    params_ref[...] = jnp.concatenate(
        [ca, cb, cc, px, py, op_eff, col[0], col[1], col[2], zc,
         zero, zero, zero, zero, zero, zero], axis=0)


def _sort_kernel(params_ref, zcol_ref, sorted_ref, rank_ref):
    zcol = zcol_ref[...]
    zrow = params_ref[9:10, :]

    rank = jnp.zeros((N, 1), jnp.float32)
    for c in range(NCHUNK):
        sl = slice(c * CHUNK, (c + 1) * CHUNK)
        zi = jnp.broadcast_to(zcol, (N, CHUNK))
        zj = jnp.broadcast_to(zrow[:, sl], (N, CHUNK))
        ii = jax.lax.broadcasted_iota(jnp.int32, (N, CHUNK), 0)
        jj = jax.lax.broadcasted_iota(jnp.int32, (N, CHUNK), 1) + c * CHUNK
        before = (zj < zi) | ((zj == zi) & (jj < ii))
        rank = rank + jnp.sum(before.astype(jnp.float32), axis=1, keepdims=True)
    rank_ref[...] = rank

    params = params_ref[...]
    p_hi, p_lo = _split2(params)
    for c in range(NCHUNK):
        kidx = (jax.lax.broadcasted_iota(jnp.int32, (N, CHUNK), 1)
                + c * CHUNK).astype(jnp.float32)
        onehot = (rank == kidx).astype(jnp.float32)
        sorted_ref[:, c * CHUNK:(c + 1) * CHUNK] = (
            jnp.dot(p_hi, onehot, preferred_element_type=jnp.float32)
            + jnp.dot(p_lo, onehot, preferred_element_type=jnp.float32))


def _raster_kernel(sorted_ref, sortedT_ref, bg_ref, img_ref, nt_ref):
    t = pl.program_id(0)

    @pl.when(t == 0)
    def _init():
        nt_ref[...] = jnp.zeros_like(nt_ref)

    pix = jax.lax.broadcasted_iota(jnp.int32, (PIX_TILE, 1), 0) + t * PIX_TILE
    gx = (pix % W).astype(jnp.float32)
    gy = (pix // W).astype(jnp.float32)

    rj = jax.lax.broadcasted_iota(jnp.int32, (CHUNK, CHUNK), 0)
    rk = jax.lax.broadcasted_iota(jnp.int32, (CHUNK, CHUNK), 1)
    ut = (rj < rk).astype(jnp.float32)

    tcarry = jnp.ones((PIX_TILE, 1), jnp.float32)
    rgbd = jnp.zeros((PIX_TILE, 4), jnp.float32)
    for c in range(NCHUNK):
        sl = slice(c * CHUNK, (c + 1) * CHUNK)
        ca = sorted_ref[0:1, sl]
        cb = sorted_ref[1:2, sl]
        cc = sorted_ref[2:3, sl]
        px = sorted_ref[3:4, sl]
        py = sorted_ref[4:5, sl]
        op = sorted_ref[5:6, sl]
        c4 = sortedT_ref[sl, 6:10]

        dx = px - gx
        dy = py - gy
        power = -0.5 * (ca * dx * dx + cc * dy * dy) - cb * dx * dy
        g = jnp.exp(jnp.minimum(power, 0.0))
        alpha = jnp.minimum(0.99, op * g)
        valid = (power <= 0.0) & (alpha >= 1.0 / 255.0)
        alpha = jnp.where(valid, alpha, 0.0)

        logm = jnp.log1p(-alpha)
        l_hi, l_lo = _split2(logm)
        sex = (jnp.dot(l_hi, ut, preferred_element_type=jnp.float32)
               + jnp.dot(l_lo, ut, preferred_element_type=jnp.float32))
        w = alpha * (tcarry * jnp.exp(sex))
        w_hi, w_lo = _split2(w)
        c4_hi, c4_lo = _split2(c4)
        rgbd = rgbd + (
            jnp.dot(w_hi, c4_hi, preferred_element_type=jnp.float32)
            + jnp.dot(w_hi, c4_lo, preferred_element_type=jnp.float32)
            + jnp.dot(w_lo, c4_hi, preferred_element_type=jnp.float32))
        tcarry = tcarry * jnp.exp(sex[:, CHUNK - 1:CHUNK] + logm[:, CHUNK - 1:CHUNK])

        nt_ref[0:1, sl] += jnp.sum(
            (alpha > 0.0).astype(jnp.float32), axis=0, keepdims=True)

    bg0 = bg_ref[0, 0]
    bg1 = bg_ref[0, 1]
    bg2 = bg_ref[0, 2]
    zeros3 = jnp.zeros((PIX_TILE, 3), jnp.float32)
    img_ref[...] = jnp.concatenate(
        [rgbd[:, 0:1] + tcarry * bg0,
         rgbd[:, 1:2] + tcarry * bg1,
         rgbd[:, 2:3] + tcarry * bg2,
         rgbd[:, 3:4],
         1.0 - tcarry,
         zeros3], axis=1)


def _unpermute_kernel(rank_ref, nt_ref, out_ref):
    rank = rank_ref[...]
    acc = jnp.zeros((N, 1), jnp.float32)
    for c in range(NCHUNK):
        kidx = (jax.lax.broadcasted_iota(jnp.int32, (N, CHUNK), 1)
                + c * CHUNK).astype(jnp.float32)
        eq = (rank == kidx).astype(jnp.float32)
        ntc = nt_ref[0:1, c * CHUNK:(c + 1) * CHUNK]
        acc = acc + jnp.sum(eq * ntc, axis=1, keepdims=True)
    out_ref[...] = acc.astype(jnp.int32)


@jax.jit
def kernel(means3D, means2D, opacities, colors_precomp, scales, rotations,
           theta, rho, viewmatrix, bg):
    del means2D, theta, rho
    pin = jnp.concatenate(
        [means3D.T.astype(jnp.float32),
         scales.T.astype(jnp.float32),
         rotations.T.astype(jnp.float32),
         opacities.T.astype(jnp.float32),
         colors_precomp.T.astype(jnp.float32),
         jnp.zeros((2, N), jnp.float32)], axis=0)

    vm32 = viewmatrix.astype(jnp.float32)
    params, radii2 = pl.pallas_call(
        _geom_kernel,
        out_shape=(
            jax.ShapeDtypeStruct((16, N), jnp.float32),
            jax.ShapeDtypeStruct((1, N), jnp.int32),
        ),
        in_specs=[
            pl.BlockSpec((16, N), lambda: (0, 0)),
            pl.BlockSpec(memory_space=pltpu.SMEM),
            pl.BlockSpec(memory_space=pltpu.SMEM),
        ],
    )(pin, vm32, _rp(vm32))

    zcol = params[9, :].reshape(N, 1)
    sorted_params, rank = pl.pallas_call(
        _sort_kernel,
        out_shape=(
            jax.ShapeDtypeStruct((16, N), jnp.float32),
            jax.ShapeDtypeStruct((N, 1), jnp.float32),
        ),
    )(params, zcol)

    img, nt_sorted = pl.pallas_call(
        _raster_kernel,
        grid=(NPIX // PIX_TILE,),
        out_shape=(
            jax.ShapeDtypeStruct((NPIX, 8), jnp.float32),
            jax.ShapeDtypeStruct((1, N), jnp.float32),
        ),
        in_specs=[
            pl.BlockSpec((16, N), lambda i: (0, 0)),
            pl.BlockSpec((N, 16), lambda i: (0, 0)),
            pl.BlockSpec(memory_space=pltpu.SMEM),
        ],
        out_specs=(
            pl.BlockSpec((PIX_TILE, 8), lambda i: (i, 0)),
            pl.BlockSpec((1, N), lambda i: (0, 0)),
        ),
    )(sorted_params, sorted_params.T, bg.reshape(1, 3).astype(jnp.float32))

    nt = pl.pallas_call(
        _unpermute_kernel,
        out_shape=jax.ShapeDtypeStruct((N, 1), jnp.int32),
    )(rank, nt_sorted)

    color = img[:, 0:3].T.reshape(3, H, W)
    depth = img[:, 3].reshape(1, H, W)
    opac = img[:, 4].reshape(1, H, W)
    return color, radii2.reshape(N), depth, opac, nt.reshape(N)


# speedup vs baseline: 3.2306x; 1.0839x over previous
import functools

import jax
import jax.numpy as jnp
from jax.experimental import pallas as pl
from jax.experimental.pallas import tpu as pltpu

H = 64
W = 64
N = 2048
TANX = 0.5
TANY = 0.5
SCALE_MOD = 1.0

CHUNK = 256
PIX_TILE = 512
NCHUNK = N // CHUNK
NPIX = H * W

_HI = jax.lax.Precision.HIGHEST


def _split2(x):
    hi = _rp(x)
    return hi, x - hi


def _rp(x):
    i = jax.lax.bitcast_convert_type(x, jnp.int32)
    i = i + jnp.int32(0x7FFF) + ((i >> 16) & jnp.int32(1))
    i = i & jnp.int32(-65536)
    return jax.lax.bitcast_convert_type(i, jnp.float32)


def _prep_kernel(pin_ref, vm_ref, vmr_ref, sorted_ref, radii_ref, rank_ref):
    def row(i):
        return pin_ref[i:i + 1, :]

    m0, m1, m2 = row(0), row(1), row(2)
    s = [row(3), row(4), row(5)]
    qr, qx, qy, qz = row(6), row(7), row(8), row(9)
    opac = row(10)
    col = [row(11), row(12), row(13)]

    qn = jnp.sqrt(qr * qr + qx * qx + qy * qy + qz * qz) + 1e-8
    r, x, y, z = qr / qn, qx / qn, qy / qn, qz / qn
    R = [
        [1 - 2 * (y * y + z * z), 2 * (x * y - r * z), 2 * (x * z + r * y)],
        [2 * (x * y + r * z), 1 - 2 * (x * x + z * z), 2 * (y * z - r * x)],
        [2 * (x * z - r * y), 2 * (y * z + r * x), 1 - 2 * (x * x + y * y)],
    ]
    L = [[_rp(R[a][c] * (s[c] * SCALE_MOD)) for c in range(3)] for a in range(3)]
    Sig = [[L[a][0] * L[b][0] + L[a][1] * L[b][1] + L[a][2] * L[b][2]
            for b in range(3)] for a in range(3)]

    V = [[vm_ref[a, b] for b in range(4)] for a in range(4)]
    Vr = [[vmr_ref[a, b] for b in range(4)] for a in range(4)]
    mr = [_rp(m0), _rp(m1), _rp(m2)]
    xv = mr[0] * Vr[0][0] + mr[1] * Vr[1][0] + mr[2] * Vr[2][0] + V[3][0]
    yv = mr[0] * Vr[0][1] + mr[1] * Vr[1][1] + mr[2] * Vr[2][1] + V[3][1]
    zv = mr[0] * Vr[0][2] + mr[1] * Vr[1][2] + mr[2] * Vr[2][2] + V[3][2]

    mask = zv > 0.2
    zc = jnp.maximum(zv, 0.2)
    fx = W / (2.0 * TANX)
    fy = H / (2.0 * TANY)
    limx = 1.3 * TANX
    limy = 1.3 * TANY
    tx = jnp.clip(xv / zc, -limx, limx) * zc
    ty = jnp.clip(yv / zc, -limy, limy) * zc
    j00 = fx / zc
    j02 = -fx * tx / (zc * zc)
    j11 = fy / zc
    j12 = -fy * ty / (zc * zc)

    Sr = [[_rp(Sig[a][b]) for b in range(3)] for a in range(3)]
    tmp = [[Vr[0][i] * Sr[0][k] + Vr[1][i] * Sr[1][k] + Vr[2][i] * Sr[2][k]
            for k in range(3)] for i in range(3)]
    tr = [[_rp(tmp[i][k]) for k in range(3)] for i in range(3)]
    M = [[tr[i][0] * Vr[0][l] + tr[i][1] * Vr[1][l] + tr[i][2] * Vr[2][l]
          for l in range(3)] for i in range(3)]

    Jr = [[_rp(j00), None, _rp(j02)], [None, _rp(j11), _rp(j12)]]
    Mr = [[_rp(M[a][b]) for b in range(3)] for a in range(3)]
    tmp2 = [[Jr[0][0] * Mr[0][k] + Jr[0][2] * Mr[2][k] for k in range(3)],
            [Jr[1][1] * Mr[1][k] + Jr[1][2] * Mr[2][k] for k in range(3)]]
    t2r = [[_rp(tmp2[i][k]) for k in range(3)] for i in range(2)]
    cov00 = t2r[0][0] * Jr[0][0] + t2r[0][2] * Jr[0][2]
    cov01 = t2r[0][1] * Jr[1][1] + t2r[0][2] * Jr[1][2]
    cov11 = t2r[1][1] * Jr[1][1] + t2r[1][2] * Jr[1][2]

    a_ = cov00 + 0.3
    b_ = cov01
    d_ = cov11 + 0.3
    det = jnp.maximum(a_ * d_ - b_ * b_, 1e-8)
    ca = d_ / det
    cb = -b_ / det
    cc = a_ / det
    mid = 0.5 * (a_ + d_)
    lam1 = mid + jnp.sqrt(jnp.maximum(mid * mid - det, 0.1))
    radii_ref[0:1, :] = jnp.where(
        mask, jnp.ceil(3.0 * jnp.sqrt(lam1)), 0.0).astype(jnp.int32)

    px = ((xv / zc) / TANX + 1.0) * W * 0.5 - 0.5
    py = ((yv / zc) / TANY + 1.0) * H * 0.5 - 0.5
    op_eff = jnp.where(mask, opac, 0.0)

    zero = jnp.zeros((1, N), jnp.float32)
    params = jnp.concatenate(
        [ca, cb, cc, px, py, op_eff, col[0], col[1], col[2], zc,
         zero, zero, zero, zero, zero, zero], axis=0)

    ones11 = jnp.ones((1, 1), jnp.float32)
    dn = (((0,), (0,)), ((), ()))
    z1 = _rp(zc)
    r1 = zc - z1
    z2 = _rp(r1)
    z3 = r1 - z2
    zcol = (jax.lax.dot_general(z1, ones11, dn, preferred_element_type=jnp.float32)
            + jax.lax.dot_general(z2, ones11, dn, preferred_element_type=jnp.float32)
            + jax.lax.dot_general(z3, ones11, dn, preferred_element_type=jnp.float32))
    zrow = zc

    rank = jnp.zeros((N, 1), jnp.float32)
    for c in range(NCHUNK):
        sl = slice(c * CHUNK, (c + 1) * CHUNK)
        zi = jnp.broadcast_to(zcol, (N, CHUNK))
        zj = jnp.broadcast_to(zrow[:, sl], (N, CHUNK))
        ii = jax.lax.broadcasted_iota(jnp.int32, (N, CHUNK), 0)
        jj = jax.lax.broadcasted_iota(jnp.int32, (N, CHUNK), 1) + c * CHUNK
        before = (zj < zi) | ((zj == zi) & (jj < ii))
        rank = rank + jnp.sum(before.astype(jnp.float32), axis=1, keepdims=True)
    rank_ref[...] = rank

    p_hi, p_lo = _split2(params)
    for c in range(NCHUNK):
        kidx = (jax.lax.broadcasted_iota(jnp.int32, (N, CHUNK), 1)
                + c * CHUNK).astype(jnp.float32)
        onehot = (rank == kidx).astype(jnp.float32)
        sorted_ref[:, c * CHUNK:(c + 1) * CHUNK] = (
            jnp.dot(p_hi, onehot, preferred_element_type=jnp.float32)
            + jnp.dot(p_lo, onehot, preferred_element_type=jnp.float32))


def _raster_kernel(sorted_ref, sortedT_ref, bg_ref, rank_ref, img_ref,
                   nt_ref, ntout_ref):
    t = pl.program_id(0)

    @pl.when(t == 0)
    def _init():
        nt_ref[...] = jnp.zeros_like(nt_ref)

    @pl.when(t == NPIX // PIX_TILE)
    def _unpermute():
        rank = rank_ref[...]
        acc = jnp.zeros((N, 1), jnp.float32)
        for c in range(NCHUNK):
            kidx = (jax.lax.broadcasted_iota(jnp.int32, (N, CHUNK), 1)
                    + c * CHUNK).astype(jnp.float32)
            eq = (rank == kidx).astype(jnp.float32)
            ntc = nt_ref[0:1, c * CHUNK:(c + 1) * CHUNK]
            acc = acc + jnp.sum(eq * ntc, axis=1, keepdims=True)
        ntout_ref[...] = acc.astype(jnp.int32)

    @pl.when(t < NPIX // PIX_TILE)
    def _raster():
        _raster_tile(t, sorted_ref, sortedT_ref, bg_ref, img_ref, nt_ref)


def _raster_tile(t, sorted_ref, sortedT_ref, bg_ref, img_ref, nt_ref):
    pix = jax.lax.broadcasted_iota(jnp.int32, (PIX_TILE, 1), 0) + t * PIX_TILE
    gx = (pix % W).astype(jnp.float32)
    gy = (pix // W).astype(jnp.float32)

    rj = jax.lax.broadcasted_iota(jnp.int32, (CHUNK, CHUNK), 0)
    rk = jax.lax.broadcasted_iota(jnp.int32, (CHUNK, CHUNK), 1)
    ut = (rj < rk).astype(jnp.float32)

    tcarry = jnp.ones((PIX_TILE, 1), jnp.float32)
    rgbd = jnp.zeros((PIX_TILE, 4), jnp.float32)
    for c in range(NCHUNK):
        sl = slice(c * CHUNK, (c + 1) * CHUNK)
        ca = sorted_ref[0:1, sl]
        cb = sorted_ref[1:2, sl]
        cc = sorted_ref[2:3, sl]
        px = sorted_ref[3:4, sl]
        py = sorted_ref[4:5, sl]
        op = sorted_ref[5:6, sl]
        c4 = sortedT_ref[sl, 6:10]

        dx = px - gx
        dy = py - gy
        power = -0.5 * (ca * dx * dx + cc * dy * dy) - cb * dx * dy
        g = jnp.exp(jnp.minimum(power, 0.0))
        alpha = jnp.minimum(0.99, op * g)
        valid = (power <= 0.0) & (alpha >= 1.0 / 255.0)
        alpha = jnp.where(valid, alpha, 0.0)

        logm = jnp.log1p(-alpha)
        l_hi, l_lo = _split2(logm)
        sex = (jnp.dot(l_hi, ut, preferred_element_type=jnp.float32)
               + jnp.dot(l_lo, ut, preferred_element_type=jnp.float32))
        w = alpha * (tcarry * jnp.exp(sex))
        w_hi, w_lo = _split2(w)
        c4_hi, c4_lo = _split2(c4)
        rgbd = rgbd + (
            jnp.dot(w_hi, c4_hi, preferred_element_type=jnp.float32)
            + jnp.dot(w_hi, c4_lo, preferred_element_type=jnp.float32)
            + jnp.dot(w_lo, c4_hi, preferred_element_type=jnp.float32))
        tcarry = tcarry * jnp.exp(sex[:, CHUNK - 1:CHUNK] + logm[:, CHUNK - 1:CHUNK])

        nt_ref[0:1, sl] += jnp.sum(
            (alpha > 0.0).astype(jnp.float32), axis=0, keepdims=True)

    bg0 = bg_ref[0, 0]
    bg1 = bg_ref[0, 1]
    bg2 = bg_ref[0, 2]
    zeros3 = jnp.zeros((PIX_TILE, 3), jnp.float32)
    img_ref[...] = jnp.concatenate(
        [rgbd[:, 0:1] + tcarry * bg0,
         rgbd[:, 1:2] + tcarry * bg1,
         rgbd[:, 2:3] + tcarry * bg2,
         rgbd[:, 3:4],
         1.0 - tcarry,
         zeros3], axis=1)


@jax.jit
def kernel(means3D, means2D, opacities, colors_precomp, scales, rotations,
           theta, rho, viewmatrix, bg):
    del means2D, theta, rho
    pin = jnp.concatenate(
        [means3D.T.astype(jnp.float32),
         scales.T.astype(jnp.float32),
         rotations.T.astype(jnp.float32),
         opacities.T.astype(jnp.float32),
         colors_precomp.T.astype(jnp.float32),
         jnp.zeros((2, N), jnp.float32)], axis=0)

    vm32 = viewmatrix.astype(jnp.float32)
    NT = NPIX // PIX_TILE
    sorted_params, radii2, rank = pl.pallas_call(
        _prep_kernel,
        out_shape=(
            jax.ShapeDtypeStruct((16, N), jnp.float32),
            jax.ShapeDtypeStruct((1, N), jnp.int32),
            jax.ShapeDtypeStruct((N, 1), jnp.float32),
        ),
        in_specs=[
            pl.BlockSpec((16, N), lambda: (0, 0)),
            pl.BlockSpec(memory_space=pltpu.SMEM),
            pl.BlockSpec(memory_space=pltpu.SMEM),
        ],
    )(pin, vm32, _rp(vm32))

    img, nt_sorted, nt = pl.pallas_call(
        _raster_kernel,
        grid=(NT + 1,),
        out_shape=(
            jax.ShapeDtypeStruct((NPIX, 8), jnp.float32),
            jax.ShapeDtypeStruct((1, N), jnp.float32),
            jax.ShapeDtypeStruct((N, 1), jnp.int32),
        ),
        in_specs=[
            pl.BlockSpec((16, N), lambda i: (0, 0)),
            pl.BlockSpec((N, 16), lambda i: (0, 0)),
            pl.BlockSpec(memory_space=pltpu.SMEM),
            pl.BlockSpec((N, 1), lambda i: (0, 0)),
        ],
        out_specs=(
            pl.BlockSpec((PIX_TILE, 8), lambda i: (jnp.minimum(i, NT - 1), 0)),
            pl.BlockSpec((1, N), lambda i: (0, 0)),
            pl.BlockSpec((N, 1), lambda i: (0, 0)),
        ),
    )(sorted_params, sorted_params.T, bg.reshape(1, 3).astype(jnp.float32),
      rank)

    color = img[:, 0:3].T.reshape(3, H, W)
    depth = img[:, 3].reshape(1, H, W)
    opac = img[:, 4].reshape(1, H, W)
    return color, radii2.reshape(N), depth, opac, nt.reshape(N)


# speedup vs baseline: 3.3146x; 1.0260x over previous
import functools

import jax
import jax.numpy as jnp
from jax.experimental import pallas as pl
from jax.experimental.pallas import tpu as pltpu

H = 64
W = 64
N = 2048
TANX = 0.5
TANY = 0.5
SCALE_MOD = 1.0

CHUNK = 256
PIX_TILE = 1024
NCHUNK = N // CHUNK
NPIX = H * W

_HI = jax.lax.Precision.HIGHEST


def _split2(x):
    hi = _rp(x)
    return hi, x - hi


def _rp(x):
    i = jax.lax.bitcast_convert_type(x, jnp.int32)
    i = i + jnp.int32(0x7FFF) + ((i >> 16) & jnp.int32(1))
    i = i & jnp.int32(-65536)
    return jax.lax.bitcast_convert_type(i, jnp.float32)


def _prep_kernel(pin_ref, vm_ref, vmr_ref, sorted_ref, radii_ref, rank_ref):
    def row(i):
        return pin_ref[i:i + 1, :]

    m0, m1, m2 = row(0), row(1), row(2)
    s = [row(3), row(4), row(5)]
    qr, qx, qy, qz = row(6), row(7), row(8), row(9)
    opac = row(10)
    col = [row(11), row(12), row(13)]

    qn = jnp.sqrt(qr * qr + qx * qx + qy * qy + qz * qz) + 1e-8
    r, x, y, z = qr / qn, qx / qn, qy / qn, qz / qn
    R = [
        [1 - 2 * (y * y + z * z), 2 * (x * y - r * z), 2 * (x * z + r * y)],
        [2 * (x * y + r * z), 1 - 2 * (x * x + z * z), 2 * (y * z - r * x)],
        [2 * (x * z - r * y), 2 * (y * z + r * x), 1 - 2 * (x * x + y * y)],
    ]
    L = [[_rp(R[a][c] * (s[c] * SCALE_MOD)) for c in range(3)] for a in range(3)]
    Sig = [[L[a][0] * L[b][0] + L[a][1] * L[b][1] + L[a][2] * L[b][2]
            for b in range(3)] for a in range(3)]

    V = [[vm_ref[a, b] for b in range(4)] for a in range(4)]
    Vr = [[vmr_ref[a, b] for b in range(4)] for a in range(4)]
    mr = [_rp(m0), _rp(m1), _rp(m2)]
    xv = mr[0] * Vr[0][0] + mr[1] * Vr[1][0] + mr[2] * Vr[2][0] + V[3][0]
    yv = mr[0] * Vr[0][1] + mr[1] * Vr[1][1] + mr[2] * Vr[2][1] + V[3][1]
    zv = mr[0] * Vr[0][2] + mr[1] * Vr[1][2] + mr[2] * Vr[2][2] + V[3][2]

    mask = zv > 0.2
    zc = jnp.maximum(zv, 0.2)
    fx = W / (2.0 * TANX)
    fy = H / (2.0 * TANY)
    limx = 1.3 * TANX
    limy = 1.3 * TANY
    tx = jnp.clip(xv / zc, -limx, limx) * zc
    ty = jnp.clip(yv / zc, -limy, limy) * zc
    j00 = fx / zc
    j02 = -fx * tx / (zc * zc)
    j11 = fy / zc
    j12 = -fy * ty / (zc * zc)

    Sr = [[_rp(Sig[a][b]) for b in range(3)] for a in range(3)]
    tmp = [[Vr[0][i] * Sr[0][k] + Vr[1][i] * Sr[1][k] + Vr[2][i] * Sr[2][k]
            for k in range(3)] for i in range(3)]
    tr = [[_rp(tmp[i][k]) for k in range(3)] for i in range(3)]
    M = [[tr[i][0] * Vr[0][l] + tr[i][1] * Vr[1][l] + tr[i][2] * Vr[2][l]
          for l in range(3)] for i in range(3)]

    Jr = [[_rp(j00), None, _rp(j02)], [None, _rp(j11), _rp(j12)]]
    Mr = [[_rp(M[a][b]) for b in range(3)] for a in range(3)]
    tmp2 = [[Jr[0][0] * Mr[0][k] + Jr[0][2] * Mr[2][k] for k in range(3)],
            [Jr[1][1] * Mr[1][k] + Jr[1][2] * Mr[2][k] for k in range(3)]]
    t2r = [[_rp(tmp2[i][k]) for k in range(3)] for i in range(2)]
    cov00 = t2r[0][0] * Jr[0][0] + t2r[0][2] * Jr[0][2]
    cov01 = t2r[0][1] * Jr[1][1] + t2r[0][2] * Jr[1][2]
    cov11 = t2r[1][1] * Jr[1][1] + t2r[1][2] * Jr[1][2]

    a_ = cov00 + 0.3
    b_ = cov01
    d_ = cov11 + 0.3
    det = jnp.maximum(a_ * d_ - b_ * b_, 1e-8)
    ca = d_ / det
    cb = -b_ / det
    cc = a_ / det
    mid = 0.5 * (a_ + d_)
    lam1 = mid + jnp.sqrt(jnp.maximum(mid * mid - det, 0.1))
    radii_ref[0:1, :] = jnp.where(
        mask, jnp.ceil(3.0 * jnp.sqrt(lam1)), 0.0).astype(jnp.int32)

    px = ((xv / zc) / TANX + 1.0) * W * 0.5 - 0.5
    py = ((yv / zc) / TANY + 1.0) * H * 0.5 - 0.5
    op_eff = jnp.where(mask, opac, 0.0)

    zero = jnp.zeros((1, N), jnp.float32)
    params = jnp.concatenate(
        [ca, cb, cc, px, py, op_eff, col[0], col[1], col[2], zc,
         zero, zero, zero, zero, zero, zero], axis=0)

    ones11 = jnp.ones((1, 1), jnp.float32)
    dn = (((0,), (0,)), ((), ()))
    z1 = _rp(zc)
    r1 = zc - z1
    z2 = _rp(r1)
    z3 = r1 - z2
    zcol = (jax.lax.dot_general(z1, ones11, dn, preferred_element_type=jnp.float32)
            + jax.lax.dot_general(z2, ones11, dn, preferred_element_type=jnp.float32)
            + jax.lax.dot_general(z3, ones11, dn, preferred_element_type=jnp.float32))
    zrow = zc

    rank = jnp.zeros((N, 1), jnp.float32)
    for c in range(NCHUNK):
        sl = slice(c * CHUNK, (c + 1) * CHUNK)
        zi = jnp.broadcast_to(zcol, (N, CHUNK))
        zj = jnp.broadcast_to(zrow[:, sl], (N, CHUNK))
        ii = jax.lax.broadcasted_iota(jnp.int32, (N, CHUNK), 0)
        jj = jax.lax.broadcasted_iota(jnp.int32, (N, CHUNK), 1) + c * CHUNK
        before = (zj < zi) | ((zj == zi) & (jj < ii))
        rank = rank + jnp.sum(before.astype(jnp.float32), axis=1, keepdims=True)
    rank_ref[...] = rank

    p_hi, p_lo = _split2(params)
    for c in range(NCHUNK):
        kidx = (jax.lax.broadcasted_iota(jnp.int32, (N, CHUNK), 1)
                + c * CHUNK).astype(jnp.float32)
        onehot = (rank == kidx).astype(jnp.float32)
        sorted_ref[:, c * CHUNK:(c + 1) * CHUNK] = (
            jnp.dot(p_hi, onehot, preferred_element_type=jnp.float32)
            + jnp.dot(p_lo, onehot, preferred_element_type=jnp.float32))


def _raster_kernel(sorted_ref, sortedT_ref, bg_ref, rank_ref, img_ref,
                   nt_ref, ntout_ref):
    t = pl.program_id(0)

    @pl.when(t == 0)
    def _init():
        nt_ref[...] = jnp.zeros_like(nt_ref)

    @pl.when(t == NPIX // PIX_TILE)
    def _unpermute():
        rank = rank_ref[...]
        acc = jnp.zeros((N, 1), jnp.float32)
        for c in range(NCHUNK):
            kidx = (jax.lax.broadcasted_iota(jnp.int32, (N, CHUNK), 1)
                    + c * CHUNK).astype(jnp.float32)
            eq = (rank == kidx).astype(jnp.float32)
            ntc = nt_ref[0:1, c * CHUNK:(c + 1) * CHUNK]
            acc = acc + jnp.sum(eq * ntc, axis=1, keepdims=True)
        ntout_ref[...] = acc.astype(jnp.int32)

    @pl.when(t < NPIX // PIX_TILE)
    def _raster():
        _raster_tile(t, sorted_ref, sortedT_ref, bg_ref, img_ref, nt_ref)


def _raster_tile(t, sorted_ref, sortedT_ref, bg_ref, img_ref, nt_ref):
    pix = jax.lax.broadcasted_iota(jnp.int32, (PIX_TILE, 1), 0) + t * PIX_TILE
    gx = (pix % W).astype(jnp.float32)
    gy = (pix // W).astype(jnp.float32)

    rj = jax.lax.broadcasted_iota(jnp.int32, (CHUNK, CHUNK), 0)
    rk = jax.lax.broadcasted_iota(jnp.int32, (CHUNK, CHUNK), 1)
    ut = (rj < rk).astype(jnp.float32)

    tcarry = jnp.ones((PIX_TILE, 1), jnp.float32)
    rgbd = jnp.zeros((PIX_TILE, 4), jnp.float32)
    for c in range(NCHUNK):
        sl = slice(c * CHUNK, (c + 1) * CHUNK)
        ca = sorted_ref[0:1, sl]
        cb = sorted_ref[1:2, sl]
        cc = sorted_ref[2:3, sl]
        px = sorted_ref[3:4, sl]
        py = sorted_ref[4:5, sl]
        op = sorted_ref[5:6, sl]
        c4 = sortedT_ref[sl, 6:10]

        dx = px - gx
        dy = py - gy
        power = -0.5 * (ca * dx * dx + cc * dy * dy) - cb * dx * dy
        g = jnp.exp(jnp.minimum(power, 0.0))
        alpha = jnp.minimum(0.99, op * g)
        valid = (power <= 0.0) & (alpha >= 1.0 / 255.0)
        alpha = jnp.where(valid, alpha, 0.0)

        logm = jnp.log1p(-alpha)
        l_hi, l_lo = _split2(logm)
        sex = (jnp.dot(l_hi, ut, preferred_element_type=jnp.float32)
               + jnp.dot(l_lo, ut, preferred_element_type=jnp.float32))
        w = alpha * (tcarry * jnp.exp(sex))
        w_hi, w_lo = _split2(w)
        c4_hi, c4_lo = _split2(c4)
        rgbd = rgbd + (
            jnp.dot(w_hi, c4_hi, preferred_element_type=jnp.float32)
            + jnp.dot(w_hi, c4_lo, preferred_element_type=jnp.float32)
            + jnp.dot(w_lo, c4_hi, preferred_element_type=jnp.float32))
        tcarry = tcarry * jnp.exp(sex[:, CHUNK - 1:CHUNK] + logm[:, CHUNK - 1:CHUNK])

        nt_ref[0:1, sl] += jnp.sum(
            valid.astype(jnp.float32), axis=0, keepdims=True)

    bg0 = bg_ref[0, 0]
    bg1 = bg_ref[0, 1]
    bg2 = bg_ref[0, 2]
    zeros3 = jnp.zeros((PIX_TILE, 3), jnp.float32)
    img_ref[...] = jnp.concatenate(
        [rgbd[:, 0:1] + tcarry * bg0,
         rgbd[:, 1:2] + tcarry * bg1,
         rgbd[:, 2:3] + tcarry * bg2,
         rgbd[:, 3:4],
         1.0 - tcarry,
         zeros3], axis=1)


@jax.jit
def kernel(means3D, means2D, opacities, colors_precomp, scales, rotations,
           theta, rho, viewmatrix, bg):
    del means2D, theta, rho
    pin = jnp.concatenate(
        [means3D.T.astype(jnp.float32),
         scales.T.astype(jnp.float32),
         rotations.T.astype(jnp.float32),
         opacities.T.astype(jnp.float32),
         colors_precomp.T.astype(jnp.float32),
         jnp.zeros((2, N), jnp.float32)], axis=0)

    vm32 = viewmatrix.astype(jnp.float32)
    NT = NPIX // PIX_TILE
    sorted_params, radii2, rank = pl.pallas_call(
        _prep_kernel,
        out_shape=(
            jax.ShapeDtypeStruct((16, N), jnp.float32),
            jax.ShapeDtypeStruct((1, N), jnp.int32),
            jax.ShapeDtypeStruct((N, 1), jnp.float32),
        ),
        in_specs=[
            pl.BlockSpec((16, N), lambda: (0, 0)),
            pl.BlockSpec(memory_space=pltpu.SMEM),
            pl.BlockSpec(memory_space=pltpu.SMEM),
        ],
    )(pin, vm32, _rp(vm32))

    img, nt_sorted, nt = pl.pallas_call(
        _raster_kernel,
        grid=(NT + 1,),
        out_shape=(
            jax.ShapeDtypeStruct((NPIX, 8), jnp.float32),
            jax.ShapeDtypeStruct((1, N), jnp.float32),
            jax.ShapeDtypeStruct((N, 1), jnp.int32),
        ),
        in_specs=[
            pl.BlockSpec((16, N), lambda i: (0, 0)),
            pl.BlockSpec((N, 16), lambda i: (0, 0)),
            pl.BlockSpec(memory_space=pltpu.SMEM),
            pl.BlockSpec((N, 1), lambda i: (0, 0)),
        ],
        out_specs=(
            pl.BlockSpec((PIX_TILE, 8), lambda i: (jnp.minimum(i, NT - 1), 0)),
            pl.BlockSpec((1, N), lambda i: (0, 0)),
            pl.BlockSpec((N, 1), lambda i: (0, 0)),
        ),
    )(sorted_params, sorted_params.T, bg.reshape(1, 3).astype(jnp.float32),
      rank)

    color = img[:, 0:3].T.reshape(3, H, W)
    depth = img[:, 3].reshape(1, H, W)
    opac = img[:, 4].reshape(1, H, W)
    return color, radii2.reshape(N), depth, opac, nt.reshape(N)
